# baseline (speedup 1.0000x reference)
import jax
import jax.numpy as jnp
from jax.experimental import pallas as pl
from jax.experimental.pallas import tpu as pltpu

N = 10000
D = 128
NH = N // 2
BMH = 40


def _att(o, w, b):
    norm = jnp.sqrt(jnp.sum(o * o, axis=1, keepdims=True)) + 1e-16
    a = jnp.sum((o / norm) * w, axis=1, keepdims=True) + b
    a = jnp.where(a > 0, a, 5.0 * (jnp.exp(jnp.minimum(a, 0.0)) - 1.0))
    return jax.nn.sigmoid(a)


def _main_body(x_ref, al0_ref, al1_ref, ah0_ref, ah1_ref,
               wl_ref, wh_ref, wm_ref,
               awl_ref, awh_ref, awm_ref, ab_ref, out_ref,
               sl_ref, sh_ref):
    i = pl.program_id(0)

    @pl.when(i == 0)
    def _():
        x = x_ref[...]
        sl_ref[...] = jnp.maximum(
            jnp.dot(x, wl_ref[...], preferred_element_type=jnp.float32), 0.0)
        sh_ref[...] = jnp.maximum(
            jnp.dot(x, wh_ref[...], preferred_element_type=jnp.float32), 0.0)

    sl = sl_ref[...]
    sh = sh_ref[...]
    wm = wm_ref[...]
    awl = awl_ref[...]
    awh = awh_ref[...]
    awm = awm_ref[...]

    def half(a_l, a_h, row0):
        o_low = jnp.dot(a_l, sl, preferred_element_type=jnp.float32)
        o_high = jnp.dot(a_h, sh, preferred_element_type=jnp.float32)
        x_blk = x_ref[pl.ds(row0, BMH), :]
        o_mlp = jnp.maximum(
            jnp.dot(x_blk, wm, preferred_element_type=jnp.float32), 0.0)
        return 3.0 * (_att(o_low, awl, ab_ref[0, 0]) * o_low
                      + _att(o_high, awh, ab_ref[0, 1]) * o_high
                      + _att(o_mlp, awm, ab_ref[0, 2]) * o_mlp)

    out_ref[0] = half(al0_ref[0], ah0_ref[0], i * BMH)
    out_ref[1] = half(al1_ref[0], ah1_ref[0], NH + i * BMH)


@jax.jit
def kernel(input, adj_low, adj_high, weight_low, weight_high, weight_mlp,
           att_w_low, att_w_high, att_w_mlp, att_b_low, att_b_high, att_b_mlp):
    awl = att_w_low.reshape(1, D)
    awh = att_w_high.reshape(1, D)
    awm = att_w_mlp.reshape(1, D)
    ab = jnp.stack([att_b_low, att_b_high, att_b_mlp], axis=1)

    al = adj_low.reshape(2, NH, N)
    ah = adj_high.reshape(2, NH, N)

    grid = (NH // BMH,)
    out = pl.pallas_call(
        _main_body,
        grid=grid,
        in_specs=[
            pl.BlockSpec((N, D), lambda i: (0, 0)),
            pl.BlockSpec((1, BMH, N), lambda i: (0, i, 0)),
            pl.BlockSpec((1, BMH, N), lambda i: (1, i, 0)),
            pl.BlockSpec((1, BMH, N), lambda i: (0, i, 0)),
            pl.BlockSpec((1, BMH, N), lambda i: (1, i, 0)),
            pl.BlockSpec((D, D), lambda i: (0, 0)),
            pl.BlockSpec((D, D), lambda i: (0, 0)),
            pl.BlockSpec((D, D), lambda i: (0, 0)),
            pl.BlockSpec((1, D), lambda i: (0, 0)),
            pl.BlockSpec((1, D), lambda i: (0, 0)),
            pl.BlockSpec((1, D), lambda i: (0, 0)),
            pl.BlockSpec((1, 3), lambda i: (0, 0)),
        ],
        out_specs=pl.BlockSpec((2, BMH, D), lambda i: (0, i, 0)),
        out_shape=jax.ShapeDtypeStruct((2, NH, D), jnp.float32),
        scratch_shapes=[
            pltpu.VMEM((N, D), jnp.float32),
            pltpu.VMEM((N, D), jnp.float32),
        ],
        compiler_params=pltpu.CompilerParams(
            dimension_semantics=("arbitrary",),
        ),
    )(input, al, al, ah, ah,
      weight_low, weight_high, weight_mlp, awl, awh, awm, ab)
    return out.reshape(N, D)


# speedup vs baseline: 1.2348x; 1.2348x over previous
import jax
import jax.numpy as jnp
from jax.experimental import pallas as pl
from jax.experimental.pallas import tpu as pltpu

N = 10000
D = 128
NH = N // 2
BMH = 104


def _att(o, w, b):
    norm = jnp.sqrt(jnp.sum(o * o, axis=1, keepdims=True)) + 1e-16
    a = jnp.sum((o / norm) * w, axis=1, keepdims=True) + b
    a = jnp.where(a > 0, a, 5.0 * (jnp.exp(jnp.minimum(a, 0.0)) - 1.0))
    return jax.nn.sigmoid(a)


def _main_body(x_ref, al0_ref, al1_ref, ah0_ref, ah1_ref,
               wl_ref, wh_ref, wm_ref,
               awl_ref, awh_ref, awm_ref, ab_ref, out_ref,
               sl_ref, sh_ref):
    i = pl.program_id(0)

    @pl.when(i == 0)
    def _():
        x = x_ref[...]
        sl_ref[...] = jnp.maximum(
            jnp.dot(x, wl_ref[...], preferred_element_type=jnp.float32), 0.0)
        sh_ref[...] = jnp.maximum(
            jnp.dot(x, wh_ref[...], preferred_element_type=jnp.float32), 0.0)

    sl = sl_ref[...]
    sh = sh_ref[...]
    wm = wm_ref[...]
    awl = awl_ref[...]
    awh = awh_ref[...]
    awm = awm_ref[...]

    def half(a_l, a_h, row0):
        o_low = jnp.dot(a_l, sl, preferred_element_type=jnp.float32)
        o_high = jnp.dot(a_h, sh, preferred_element_type=jnp.float32)
        x_blk = x_ref[pl.ds(jnp.minimum(row0, N - BMH), BMH), :]
        o_mlp = jnp.maximum(
            jnp.dot(x_blk, wm, preferred_element_type=jnp.float32), 0.0)
        return 3.0 * (_att(o_low, awl, ab_ref[0, 0]) * o_low
                      + _att(o_high, awh, ab_ref[0, 1]) * o_high
                      + _att(o_mlp, awm, ab_ref[0, 2]) * o_mlp)

    out_ref[0] = half(al0_ref[0], ah0_ref[0], i * BMH)
    out_ref[1] = half(al1_ref[0], ah1_ref[0], NH + i * BMH)


@jax.jit
def kernel(input, adj_low, adj_high, weight_low, weight_high, weight_mlp,
           att_w_low, att_w_high, att_w_mlp, att_b_low, att_b_high, att_b_mlp):
    awl = att_w_low.reshape(1, D)
    awh = att_w_high.reshape(1, D)
    awm = att_w_mlp.reshape(1, D)
    ab = jnp.stack([att_b_low, att_b_high, att_b_mlp], axis=1)

    al = adj_low.reshape(2, NH, N)
    ah = adj_high.reshape(2, NH, N)

    grid = (pl.cdiv(NH, BMH),)
    out = pl.pallas_call(
        _main_body,
        grid=grid,
        in_specs=[
            pl.BlockSpec((N, D), lambda i: (0, 0)),
            pl.BlockSpec((1, BMH, N), lambda i: (0, i, 0)),
            pl.BlockSpec((1, BMH, N), lambda i: (1, i, 0)),
            pl.BlockSpec((1, BMH, N), lambda i: (0, i, 0)),
            pl.BlockSpec((1, BMH, N), lambda i: (1, i, 0)),
            pl.BlockSpec((D, D), lambda i: (0, 0)),
            pl.BlockSpec((D, D), lambda i: (0, 0)),
            pl.BlockSpec((D, D), lambda i: (0, 0)),
            pl.BlockSpec((1, D), lambda i: (0, 0)),
            pl.BlockSpec((1, D), lambda i: (0, 0)),
            pl.BlockSpec((1, D), lambda i: (0, 0)),
            pl.BlockSpec((1, 3), lambda i: (0, 0)),
        ],
        out_specs=pl.BlockSpec((2, BMH, D), lambda i: (0, i, 0)),
        out_shape=jax.ShapeDtypeStruct((2, NH, D), jnp.float32),
        scratch_shapes=[
            pltpu.VMEM((N, D), jnp.float32),
            pltpu.VMEM((N, D), jnp.float32),
        ],
        compiler_params=pltpu.CompilerParams(
            dimension_semantics=("arbitrary",),
        ),
    )(input, al, al, ah, ah,
      weight_low, weight_high, weight_mlp, awl, awh, awm, ab)
    return out.reshape(N, D)


# speedup vs baseline: 1.2409x; 1.0049x over previous
import jax
import jax.numpy as jnp
from jax.experimental import pallas as pl
from jax.experimental.pallas import tpu as pltpu

N = 10000
D = 128
BM = 248


def _att(o, w, b):
    norm = jnp.sqrt(jnp.sum(o * o, axis=1, keepdims=True)) + 1e-16
    a = jnp.sum((o / norm) * w, axis=1, keepdims=True) + b
    a = jnp.where(a > 0, a, 5.0 * (jnp.exp(jnp.minimum(a, 0.0)) - 1.0))
    return jax.nn.sigmoid(a)


def _main_body(x_ref, al_ref, ah_ref,
               wl_ref, wh_ref, wm_ref,
               awl_ref, awh_ref, awm_ref, ab_ref, out_ref,
               sl_ref, sh_ref):
    i = pl.program_id(0)

    @pl.when(i == 0)
    def _():
        x = x_ref[...]
        sl_ref[...] = jnp.maximum(
            jnp.dot(x, wl_ref[...], preferred_element_type=jnp.float32), 0.0)
        sh_ref[...] = jnp.maximum(
            jnp.dot(x, wh_ref[...], preferred_element_type=jnp.float32), 0.0)

    o_low = jnp.dot(al_ref[...], sl_ref[...],
                    preferred_element_type=jnp.float32)
    o_high = jnp.dot(ah_ref[...], sh_ref[...],
                     preferred_element_type=jnp.float32)
    x_blk = x_ref[pl.ds(jnp.minimum(i * BM, N - BM), BM), :]
    o_mlp = jnp.maximum(
        jnp.dot(x_blk, wm_ref[...], preferred_element_type=jnp.float32), 0.0)
    att_low = _att(o_low, awl_ref[...], ab_ref[0, 0])
    att_high = _att(o_high, awh_ref[...], ab_ref[0, 1])
    att_mlp = _att(o_mlp, awm_ref[...], ab_ref[0, 2])
    out_ref[...] = 3.0 * (att_low * o_low + att_high * o_high
                          + att_mlp * o_mlp)


@jax.jit
def kernel(input, adj_low, adj_high, weight_low, weight_high, weight_mlp,
           att_w_low, att_w_high, att_w_mlp, att_b_low, att_b_high, att_b_mlp):
    awl = att_w_low.reshape(1, D)
    awh = att_w_high.reshape(1, D)
    awm = att_w_mlp.reshape(1, D)
    ab = jnp.stack([att_b_low, att_b_high, att_b_mlp], axis=1)

    grid = (pl.cdiv(N, BM),)
    out = pl.pallas_call(
        _main_body,
        grid=grid,
        in_specs=[
            pl.BlockSpec((N, D), lambda i: (0, 0)),
            pl.BlockSpec((BM, N), lambda i: (i, 0)),
            pl.BlockSpec((BM, N), lambda i: (i, 0)),
            pl.BlockSpec((D, D), lambda i: (0, 0)),
            pl.BlockSpec((D, D), lambda i: (0, 0)),
            pl.BlockSpec((D, D), lambda i: (0, 0)),
            pl.BlockSpec((1, D), lambda i: (0, 0)),
            pl.BlockSpec((1, D), lambda i: (0, 0)),
            pl.BlockSpec((1, D), lambda i: (0, 0)),
            pl.BlockSpec((1, 3), lambda i: (0, 0)),
        ],
        out_specs=pl.BlockSpec((BM, D), lambda i: (i, 0)),
        out_shape=jax.ShapeDtypeStruct((N, D), jnp.float32),
        scratch_shapes=[
            pltpu.VMEM((N, D), jnp.float32),
            pltpu.VMEM((N, D), jnp.float32),
        ],
        compiler_params=pltpu.CompilerParams(
            dimension_semantics=("arbitrary",),
        ),
    )(input, adj_low, adj_high,
      weight_low, weight_high, weight_mlp, awl, awh, awm, ab)
    return out


# speedup vs baseline: 1.2443x; 1.0027x over previous
import jax
import jax.numpy as jnp
from jax.experimental import pallas as pl
from jax.experimental.pallas import tpu as pltpu

N = 10000
D = 128
BM = 200


def _att(o, w, b):
    norm = jnp.sqrt(jnp.sum(o * o, axis=1, keepdims=True)) + 1e-16
    a = jnp.sum((o / norm) * w, axis=1, keepdims=True) + b
    a = jnp.where(a > 0, a, 5.0 * (jnp.exp(jnp.minimum(a, 0.0)) - 1.0))
    return jax.nn.sigmoid(a)


def _main_body(x_ref, al_ref, ah_ref,
               wl_ref, wh_ref, wm_ref,
               awl_ref, awh_ref, awm_ref, ab_ref, out_ref,
               sl_ref, sh_ref):
    i = pl.program_id(0)

    @pl.when(i == 0)
    def _():
        x = x_ref[...]
        sl_ref[...] = jnp.maximum(
            jnp.dot(x, wl_ref[...], preferred_element_type=jnp.float32), 0.0)
        sh_ref[...] = jnp.maximum(
            jnp.dot(x, wh_ref[...], preferred_element_type=jnp.float32), 0.0)

    o_low = jnp.dot(al_ref[...], sl_ref[...],
                    preferred_element_type=jnp.float32)
    o_high = jnp.dot(ah_ref[...], sh_ref[...],
                     preferred_element_type=jnp.float32)
    x_blk = x_ref[pl.ds(jnp.minimum(i * BM, N - BM), BM), :]
    o_mlp = jnp.maximum(
        jnp.dot(x_blk, wm_ref[...], preferred_element_type=jnp.float32), 0.0)
    att_low = _att(o_low, awl_ref[...], ab_ref[0, 0])
    att_high = _att(o_high, awh_ref[...], ab_ref[0, 1])
    att_mlp = _att(o_mlp, awm_ref[...], ab_ref[0, 2])
    out_ref[...] = 3.0 * (att_low * o_low + att_high * o_high
                          + att_mlp * o_mlp)


@jax.jit
def kernel(input, adj_low, adj_high, weight_low, weight_high, weight_mlp,
           att_w_low, att_w_high, att_w_mlp, att_b_low, att_b_high, att_b_mlp):
    awl = att_w_low.reshape(1, D)
    awh = att_w_high.reshape(1, D)
    awm = att_w_mlp.reshape(1, D)
    ab = jnp.stack([att_b_low, att_b_high, att_b_mlp], axis=1)

    grid = (pl.cdiv(N, BM),)
    out = pl.pallas_call(
        _main_body,
        grid=grid,
        in_specs=[
            pl.BlockSpec((N, D), lambda i: (0, 0)),
            pl.BlockSpec((BM, N), lambda i: (i, 0)),
            pl.BlockSpec((BM, N), lambda i: (i, 0)),
            pl.BlockSpec((D, D), lambda i: (0, 0)),
            pl.BlockSpec((D, D), lambda i: (0, 0)),
            pl.BlockSpec((D, D), lambda i: (0, 0)),
            pl.BlockSpec((1, D), lambda i: (0, 0)),
            pl.BlockSpec((1, D), lambda i: (0, 0)),
            pl.BlockSpec((1, D), lambda i: (0, 0)),
            pl.BlockSpec((1, 3), lambda i: (0, 0)),
        ],
        out_specs=pl.BlockSpec((BM, D), lambda i: (i, 0)),
        out_shape=jax.ShapeDtypeStruct((N, D), jnp.float32),
        scratch_shapes=[
            pltpu.VMEM((N, D), jnp.float32),
            pltpu.VMEM((N, D), jnp.float32),
        ],
        compiler_params=pltpu.CompilerParams(
            dimension_semantics=("arbitrary",),
        ),
    )(input, adj_low, adj_high,
      weight_low, weight_high, weight_mlp, awl, awh, awm, ab)
    return out
